# Initial kernel scaffold; baseline (speedup 1.0000x reference)
import jax
import jax.numpy as jnp
from jax.experimental import pallas as pl


def kernel(x, edge_index, edge_attr, extra_feature, selected_node_masks, original_node_mask, inter_graph_idx, params):
    raise NotImplementedError("write your pallas kernel here")


# speedup vs baseline: 2.5076x; 2.5076x over previous
import functools

import jax
import jax.numpy as jnp
from jax import lax
from jax.experimental import pallas as pl
from jax.experimental.pallas import tpu as pltpu
from jax.experimental.pallas import tpu_sc as plsc

_F32 = jnp.float32


def _enc_body(x_ref, ex_ref, wxt_ref, bx_ref, wet_ref, be_ref,
              o_ref, os_ref):
    hx = jnp.maximum(
        jnp.dot(x_ref[...], wxt_ref[...], preferred_element_type=_F32)
        + bx_ref[...][0:1, :], 0.0)
    he = jnp.maximum(
        jnp.dot(ex_ref[...], wet_ref[...], preferred_element_type=_F32)
        + be_ref[...][0:1, :], 0.0)
    h = jnp.concatenate([hx, he], axis=1)
    o_ref[...] = h
    hw = h.shape[1] // 2
    os_ref[0] = h[:, :hw]
    os_ref[1] = h[:, hw:]


def _encode(x, extra, p):
    n = x.shape[0]
    dim = p["encode_x_W"].shape[0]
    denc = p["encode_extra_W"].shape[0]
    d1 = dim + denc
    bx = jnp.broadcast_to(p["encode_x_b"][None, :], (8, dim))
    be = jnp.broadcast_to(p["encode_extra_b"][None, :], (8, denc))
    return pl.pallas_call(
        _enc_body,
        out_shape=[jax.ShapeDtypeStruct((n, d1), _F32),
                   jax.ShapeDtypeStruct((2, n, d1 // 2), _F32)],
    )(x, extra, p["encode_x_W"].T, bx, p["encode_extra_W"].T, be)


def _moments_body(ea_ref, sum_ref, gram_ref):
    i = pl.program_id(0)
    ea = ea_ref[...]
    s = jnp.sum(ea, axis=0, keepdims=True)
    g = lax.dot_general(ea, ea, (((0,), (0,)), ((), ())),
                        preferred_element_type=_F32)

    @pl.when(i == 0)
    def _():
        sum_ref[...] = jnp.zeros_like(sum_ref)
        gram_ref[...] = jnp.zeros_like(gram_ref)

    sum_ref[...] += jnp.broadcast_to(s, sum_ref.shape)
    gram_ref[...] += g


def _edge_moments(edge_attr, tile):
    e, de = edge_attr.shape
    sums, gram = pl.pallas_call(
        _moments_body,
        grid=(e // tile,),
        in_specs=[pl.BlockSpec((tile, de), lambda i: (i, 0))],
        out_specs=[pl.BlockSpec((8, de), lambda i: (0, 0)),
                   pl.BlockSpec((de, de), lambda i: (0, 0))],
        out_shape=[jax.ShapeDtypeStruct((8, de), _F32),
                   jax.ShapeDtypeStruct((de, de), _F32)],
    )(edge_attr)
    return sums[0] / e, gram / e


def _edge_mlp_body(ea_ref, w1t_ref, pp_ref, w2t_ref, y2_ref, st_ref):
    i = pl.program_id(0)
    pp = pp_ref[...]
    a1 = pp[0:1, :]
    c1 = pp[1:2, :]
    b2 = pp[2:3, :]
    y1 = jnp.dot(ea_ref[...], w1t_ref[...], preferred_element_type=_F32)
    e1 = jnp.maximum(y1 * a1 + c1, 0.0)
    y2 = jnp.dot(e1, w2t_ref[...], preferred_element_type=_F32) + b2
    hw = y2.shape[1] // 2
    y2_ref[0] = y2[:, :hw]
    y2_ref[1] = y2[:, hw:]
    s = jnp.sum(y2, axis=0, keepdims=True)
    q = jnp.sum(y2 * y2, axis=0, keepdims=True)
    st = jnp.concatenate(
        [s, q, jnp.zeros((6, y2.shape[1]), _F32)], axis=0)

    @pl.when(i == 0)
    def _():
        st_ref[...] = jnp.zeros_like(st_ref)

    st_ref[...] += st


def _edge_mlp(edge_attr, w1t, a1, c1, b2, w2t, tile):
    e, de = edge_attr.shape
    d1 = w1t.shape[1]
    pp = jnp.concatenate([a1[None, :], c1[None, :], b2[None, :],
                          jnp.zeros((5, d1), _F32)], axis=0)
    y2s, st = pl.pallas_call(
        _edge_mlp_body,
        grid=(e // tile,),
        in_specs=[pl.BlockSpec((tile, de), lambda i: (i, 0)),
                  pl.BlockSpec((de, d1), lambda i: (0, 0)),
                  pl.BlockSpec((8, d1), lambda i: (0, 0)),
                  pl.BlockSpec((d1, d1), lambda i: (0, 0))],
        out_specs=[pl.BlockSpec((2, tile, d1 // 2), lambda i: (0, i, 0)),
                   pl.BlockSpec((8, d1), lambda i: (0, 0))],
        out_shape=[jax.ShapeDtypeStruct((2, e, d1 // 2), _F32),
                   jax.ShapeDtypeStruct((8, d1), _F32)],
    )(edge_attr, w1t, pp, w2t)
    return y2s, st[0], st[1]


@functools.lru_cache(maxsize=None)
def _make_msg_kernel(e_total, n_nodes, d1):
    info = plsc.get_sparse_core_info()
    nc, ns = info.num_cores, info.num_subcores
    hw = d1 // 2
    epw = e_total // ns
    bsz = 80
    nb = epw // bsz
    nib = bsz // 16
    zr = 128
    rps = ((n_nodes // ns + zr - 1) // zr) * zr
    n_pad = rps * ns
    nlc = hw // 16
    mesh = plsc.VectorSubcoreMesh(core_axis_name="c", subcore_axis_name="s")

    @functools.partial(
        pl.kernel,
        out_type=jax.ShapeDtypeStruct((nc, n_pad, hw), _F32),
        mesh=mesh,
        compiler_params=pltpu.CompilerParams(use_tc_tiling_on_sc=False),
        scratch_types=[
            pltpu.VMEM((bsz,), jnp.int32),
            pltpu.VMEM((bsz,), jnp.int32),
            pltpu.VMEM((bsz, hw), _F32),
            pltpu.VMEM((bsz, hw), _F32),
            pltpu.VMEM((zr, hw), _F32),
            pltpu.VMEM((2, d1), _F32),
            pltpu.VMEM_SHARED((n_pad, hw), _F32),
            pltpu.SemaphoreType.DMA,
            pltpu.SemaphoreType.DMA,
        ],
    )
    def msg(y2_hbm, src_hbm, dst_hbm, h2_hbm, ac_hbm, out_hbm,
            sidx, didx, rows, y2t, zbuf, acv, aggsh, sem1, sem2):
        cid = lax.axis_index("c")
        sid = lax.axis_index("s")

        def zrow(r, carry):
            for j in range(nlc):
                zbuf[r, pl.ds(j * 16, 16)] = jnp.zeros((16,), _F32)
            return carry

        lax.fori_loop(0, zr, zrow, 0)
        for k in range(rps // zr):
            pltpu.sync_copy(zbuf, aggsh.at[pl.ds(sid * rps + k * zr, zr)])
        pltpu.sync_copy(ac_hbm, acv)
        plsc.subcore_barrier()

        a2 = [acv[0, pl.ds(cid * hw + j * 16, 16)] for j in range(nlc)]
        c2 = [acv[1, pl.ds(cid * hw + j * 16, 16)] for j in range(nlc)]
        ebase0 = sid * epw
        node_off = cid * n_nodes

        def batch_body(i, carry):
            base = ebase0 + i * bsz
            pltpu.sync_copy(src_hbm.at[pl.ds(base, bsz)], sidx)
            pltpu.sync_copy(dst_hbm.at[pl.ds(base, bsz)], didx)
            for j in range(nib):
                sl = pl.ds(j * 16, 16)
                sidx[sl] = sidx[sl] + node_off
            cp1 = pltpu.async_copy(h2_hbm.at[sidx], rows, sem1)
            cp2 = pltpu.async_copy(
                y2_hbm.at[pl.ds(cid * e_total + base, bsz)], y2t, sem2)
            cp2.wait()
            cp1.wait()

            def row_body(r, c_):
                for j in range(nlc):
                    sl = pl.ds(j * 16, 16)
                    ev = jnp.maximum(y2t[r, sl] * a2[j] + c2[j], 0.0)
                    rows[r, sl] = jnp.maximum(rows[r, sl] + ev, 0.0)
                return c_

            lax.fori_loop(0, bsz, row_body, 0)
            pltpu.sync_copy(rows, aggsh.at[didx], add=True)
            return carry

        lax.fori_loop(0, nb, batch_body, 0)
        plsc.subcore_barrier()
        for k in range(rps // zr):
            r0 = sid * rps + k * zr
            pltpu.sync_copy(aggsh.at[pl.ds(r0, zr)],
                            out_hbm.at[cid, pl.ds(r0, zr)])

    return msg


def _node_body(h_ref, agg_ref, pe_ref, w1t_ref, pp_ref, w2t_ref,
               o_ref, os_ref):
    n = h_ref.shape[0]
    pp = pp_ref[...]
    g1 = pp[0:1, :]
    beta1 = pp[1:2, :]
    g2 = pp[2:3, :]
    beta2 = pp[3:4, :]
    agg = jnp.concatenate([agg_ref[0, :n, :], agg_ref[1, :n, :]], axis=1)
    hh = h_ref[...] * pe_ref[...][0:1, :] + agg
    y1 = jnp.dot(hh, w1t_ref[...], preferred_element_type=_F32)
    m1 = jnp.mean(y1, axis=0, keepdims=True)
    v1 = jnp.mean((y1 - m1) ** 2, axis=0, keepdims=True)
    e1 = jnp.maximum(g1 * (y1 - m1) / jnp.sqrt(v1 + 1e-5) + beta1, 0.0)
    y2 = jnp.dot(e1, w2t_ref[...], preferred_element_type=_F32)
    m2 = jnp.mean(y2, axis=0, keepdims=True)
    v2 = jnp.mean((y2 - m2) ** 2, axis=0, keepdims=True)
    o = jnp.maximum(g2 * (y2 - m2) / jnp.sqrt(v2 + 1e-5) + beta2, 0.0)
    o_ref[...] = o
    hw = o.shape[1] // 2
    os_ref[0] = o[:, :hw]
    os_ref[1] = o[:, hw:]


def _node_mlp(h, agg2, c):
    n, d1 = h.shape
    dim = c["mlp_W1"].shape[0]
    pe = jnp.broadcast_to((1.0 + c["eps"][0]) * jnp.ones((1, d1), _F32),
                          (8, d1))
    pp = jnp.concatenate([c["mlp_g1"][None, :], c["mlp_beta1"][None, :],
                          c["mlp_g2"][None, :], c["mlp_beta2"][None, :],
                          jnp.zeros((4, dim), _F32)], axis=0)
    return pl.pallas_call(
        _node_body,
        out_shape=[jax.ShapeDtypeStruct((n, dim), _F32),
                   jax.ShapeDtypeStruct((2, n, dim // 2), _F32)],
    )(h, agg2, pe, c["mlp_W1"].T, pp, c["mlp_W2"].T)


def _s2s_body(h_ref, mask_ref, batch_ref, inter_ref, wiht_ref, whht_ref,
              bias_ref, fc1t_ref, fc4t_ref, o_ref, x_scr, e_scr, ex_scr,
              *, steps, nch, ngraphs, nout):
    _, cs, d = h_ref.shape
    bias = bias_ref[...]
    bgate = bias[0:1, :]
    fc1b = bias[1:2, 0:d]
    fc4b = bias[2:3, 0:1]
    iota_cb = lax.broadcasted_iota(jnp.int32, (cs, ngraphs), 1)

    def fill(k, carry):
        x_scr[k] = h_ref[k] * mask_ref[k]
        return carry

    lax.fori_loop(0, nch, fill, 0)

    q_star = jnp.zeros((ngraphs, 2 * d), _F32)
    h = jnp.zeros((ngraphs, d), _F32)
    c = jnp.zeros((ngraphs, d), _F32)

    for _ in range(steps):
        gates = (jnp.dot(q_star, wiht_ref[...], preferred_element_type=_F32)
                 + jnp.dot(h, whht_ref[...], preferred_element_type=_F32)
                 + bgate)
        ig = gates[:, 0:d]
        fg = gates[:, d:2 * d]
        gg = gates[:, 2 * d:3 * d]
        og = gates[:, 3 * d:4 * d]
        c = jax.nn.sigmoid(fg) * c + jax.nn.sigmoid(ig) * jnp.tanh(gg)
        h = jax.nn.sigmoid(og) * jnp.tanh(c)
        q = h

        def pass1(k, emax):
            oh = (batch_ref[k] == iota_cb).astype(_F32)
            qb = lax.dot_general(oh, q, (((1,), (0,)), ((), ())),
                                 preferred_element_type=_F32)
            ec = jnp.sum(x_scr[k] * qb, axis=1, keepdims=True)
            e_scr[k] = ec
            mk = jnp.max(jnp.where(oh > 0.0, ec, -1e30), axis=0,
                         keepdims=True)
            return jnp.maximum(emax, mk)

        emax = lax.fori_loop(0, nch, pass1,
                             jnp.full((1, ngraphs), -1e30, _F32))
        emax = jnp.where(emax > -1e29, emax, 0.0)

        def pass2(k, denom):
            oh = (batch_ref[k] == iota_cb).astype(_F32)
            epn = jnp.sum(oh * emax, axis=1, keepdims=True)
            ex = jnp.exp(e_scr[k] - epn)
            ex_scr[k] = ex
            return denom + jnp.sum(oh * ex, axis=0, keepdims=True)

        denom = lax.fori_loop(0, nch, pass2, jnp.zeros((1, ngraphs), _F32))

        def pass3(k, r):
            oh = (batch_ref[k] == iota_cb).astype(_F32)
            dpn = jnp.sum(oh * denom, axis=1, keepdims=True)
            a = ex_scr[k] / dpn
            ax = a * x_scr[k]
            return r + lax.dot_general(oh, ax, (((0,), (0,)), ((), ())),
                                       preferred_element_type=_F32)

        r = lax.fori_loop(0, nch, pass3, jnp.zeros((ngraphs, d), _F32))
        q_star = jnp.concatenate([q, r], axis=1)

    h1 = jnp.maximum(
        jnp.dot(q_star, fc1t_ref[...], preferred_element_type=_F32) + fc1b,
        0.0)
    iota_go = lax.broadcasted_iota(jnp.int32, (ngraphs, nout), 1)
    ohi = (inter_ref[...] == iota_go).astype(_F32)
    cnt = lax.dot_general(ohi, jnp.ones((ngraphs, 1), _F32),
                          (((0,), (0,)), ((), ())),
                          preferred_element_type=_F32)
    s = lax.dot_general(ohi, h1, (((0,), (0,)), ((), ())),
                        preferred_element_type=_F32)
    hm = s / jnp.maximum(cnt, 1.0)
    o_ref[...] = (jnp.dot(hm, fc4t_ref[...], preferred_element_type=_F32)
                  + fc4b)


def _set2set_head(h, mask, batch, inter_idx, p, ngraphs, nout):
    n, d = h.shape
    nch = 4
    cs = n // nch
    bias = jnp.zeros((8, 4 * d), _F32)
    bias = bias.at[0, :].set(p["lstm_bih"] + p["lstm_bhh"])
    bias = bias.at[1, 0:d].set(p["fc1_b"])
    bias = bias.at[2, 0].set(p["fc4_b"][0])
    body = functools.partial(_s2s_body, steps=6, nch=nch, ngraphs=ngraphs,
                             nout=nout)
    return pl.pallas_call(
        body,
        out_shape=jax.ShapeDtypeStruct((nout, 1), _F32),
        scratch_shapes=[pltpu.VMEM((nch, cs, d), _F32),
                        pltpu.VMEM((nch, cs, 1), _F32),
                        pltpu.VMEM((nch, cs, 1), _F32)],
    )(h.reshape(nch, cs, d), mask.reshape(nch, cs, 1),
      batch.reshape(nch, cs, 1),
      inter_idx.reshape(ngraphs, 1), p["lstm_Wih"].T, p["lstm_Whh"].T,
      bias, p["fc1_W"].T, p["fc4_W"].T)


def kernel(x, edge_index, edge_attr, extra_feature, selected_node_masks,
           original_node_mask, inter_graph_idx, params):
    n_nodes = x.shape[0]
    e_total = edge_attr.shape[0]
    n_graphs = inter_graph_idx.shape[0]
    n_out = 100
    src = edge_index[0]
    dst = edge_index[1]
    tile = 6400

    h, hs = _encode(x, extra_feature, params)
    mu, gram = _edge_moments(edge_attr, tile)

    for c in params["convs"]:
        w1 = c["be_W1"]
        d1 = w1.shape[0]
        wmu = w1 @ mu
        mean1 = wmu + c["be_b1"]
        ey2 = jnp.einsum("ij,jk,ik->i", w1, gram, w1) \
            + 2.0 * c["be_b1"] * wmu + c["be_b1"] ** 2
        var1 = ey2 - mean1 ** 2
        a1 = c["be_g1"] / jnp.sqrt(var1 + 1e-5)
        c1 = c["be_beta1"] - mean1 * a1

        y2s, s2, q2 = _edge_mlp(edge_attr, w1.T, a1, c1, c["be_b2"],
                                c["be_W2"].T, tile)
        mean2 = s2 / e_total
        var2 = q2 / e_total - mean2 ** 2
        a2 = c["be_g2"] / jnp.sqrt(var2 + 1e-5)
        c2 = c["be_beta2"] - mean2 * a2
        ac = jnp.stack([a2, c2], axis=0)

        msg = _make_msg_kernel(e_total, n_nodes, d1)
        agg2 = msg(y2s.reshape(2 * e_total, d1 // 2), src, dst,
                   hs.reshape(2 * n_nodes, d1 // 2), ac)
        h, hs = _node_mlp(h, agg2, c)

    return _set2set_head(h, selected_node_masks, original_node_mask,
                         inter_graph_idx, params, n_graphs, n_out)


# speedup vs baseline: 3.5644x; 1.4214x over previous
import functools

import jax
import jax.numpy as jnp
from jax import lax
from jax.experimental import pallas as pl
from jax.experimental.pallas import tpu as pltpu
from jax.experimental.pallas import tpu_sc as plsc

_F32 = jnp.float32


def _enc_body(x_ref, ex_ref, wxt_ref, bx_ref, wet_ref, be_ref,
              o_ref, os_ref):
    hx = jnp.maximum(
        jnp.dot(x_ref[...], wxt_ref[...], preferred_element_type=_F32)
        + bx_ref[...][0:1, :], 0.0)
    he = jnp.maximum(
        jnp.dot(ex_ref[...], wet_ref[...], preferred_element_type=_F32)
        + be_ref[...][0:1, :], 0.0)
    h = jnp.concatenate([hx, he], axis=1)
    o_ref[...] = h
    hw = h.shape[1] // 2
    os_ref[0] = h[:, :hw]
    os_ref[1] = h[:, hw:]


def _encode(x, extra, p):
    n = x.shape[0]
    dim = p["encode_x_W"].shape[0]
    denc = p["encode_extra_W"].shape[0]
    d1 = dim + denc
    bx = jnp.broadcast_to(p["encode_x_b"][None, :], (8, dim))
    be = jnp.broadcast_to(p["encode_extra_b"][None, :], (8, denc))
    return pl.pallas_call(
        _enc_body,
        out_shape=[jax.ShapeDtypeStruct((n, d1), _F32),
                   jax.ShapeDtypeStruct((2, n, d1 // 2), _F32)],
    )(x, extra, p["encode_x_W"].T, bx, p["encode_extra_W"].T, be)


def _moments_body(ea_ref, sum_ref, gram_ref):
    i = pl.program_id(0)
    ea = ea_ref[...]
    s = jnp.sum(ea, axis=0, keepdims=True)
    g = lax.dot_general(ea, ea, (((0,), (0,)), ((), ())),
                        preferred_element_type=_F32)

    @pl.when(i == 0)
    def _():
        sum_ref[...] = jnp.zeros_like(sum_ref)
        gram_ref[...] = jnp.zeros_like(gram_ref)

    sum_ref[...] += jnp.broadcast_to(s, sum_ref.shape)
    gram_ref[...] += g


def _edge_moments(edge_attr, tile):
    e, de = edge_attr.shape
    sums, gram = pl.pallas_call(
        _moments_body,
        grid=(e // tile,),
        in_specs=[pl.BlockSpec((tile, de), lambda i: (i, 0))],
        out_specs=[pl.BlockSpec((8, de), lambda i: (0, 0)),
                   pl.BlockSpec((de, de), lambda i: (0, 0))],
        out_shape=[jax.ShapeDtypeStruct((8, de), _F32),
                   jax.ShapeDtypeStruct((de, de), _F32)],
    )(edge_attr)
    return sums[0] / e, gram / e


def _edge_mlp_body(ea_ref, w1t_ref, pp_ref, w2t_ref, y2_ref, st_ref):
    i = pl.program_id(0)
    pp = pp_ref[...]
    a1 = pp[0:1, :]
    c1 = pp[1:2, :]
    b2 = pp[2:3, :]
    y1 = jnp.dot(ea_ref[...], w1t_ref[...], preferred_element_type=_F32)
    e1 = jnp.maximum(y1 * a1 + c1, 0.0)
    y2 = jnp.dot(e1, w2t_ref[...], preferred_element_type=_F32) + b2
    hw = y2.shape[1] // 2
    y2_ref[0] = y2[:, :hw]
    y2_ref[1] = y2[:, hw:]
    s = jnp.sum(y2, axis=0, keepdims=True)
    q = jnp.sum(y2 * y2, axis=0, keepdims=True)
    st = jnp.concatenate(
        [s, q, jnp.zeros((6, y2.shape[1]), _F32)], axis=0)

    @pl.when(i == 0)
    def _():
        st_ref[...] = jnp.zeros_like(st_ref)

    st_ref[...] += st


def _edge_mlp(edge_attr, w1t, a1, c1, b2, w2t, tile):
    e, de = edge_attr.shape
    d1 = w1t.shape[1]
    pp = jnp.concatenate([a1[None, :], c1[None, :], b2[None, :],
                          jnp.zeros((5, d1), _F32)], axis=0)
    y2s, st = pl.pallas_call(
        _edge_mlp_body,
        grid=(e // tile,),
        in_specs=[pl.BlockSpec((tile, de), lambda i: (i, 0)),
                  pl.BlockSpec((de, d1), lambda i: (0, 0)),
                  pl.BlockSpec((8, d1), lambda i: (0, 0)),
                  pl.BlockSpec((d1, d1), lambda i: (0, 0))],
        out_specs=[pl.BlockSpec((2, tile, d1 // 2), lambda i: (0, i, 0)),
                   pl.BlockSpec((8, d1), lambda i: (0, 0))],
        out_shape=[jax.ShapeDtypeStruct((2, e, d1 // 2), _F32),
                   jax.ShapeDtypeStruct((8, d1), _F32)],
    )(edge_attr, w1t, pp, w2t)
    return y2s, st[0], st[1]


@functools.lru_cache(maxsize=None)
def _make_msg_kernel(e_total, n_nodes, d1):
    info = plsc.get_sparse_core_info()
    nc, ns = info.num_cores, info.num_subcores
    hw = d1 // 2
    epw = e_total // ns
    bsz = 80
    nb = epw // bsz
    nib = bsz // 16
    zr = 128
    rps = ((n_nodes // ns + zr - 1) // zr) * zr
    n_pad = rps * ns
    nlc = hw // 16
    mesh = plsc.VectorSubcoreMesh(core_axis_name="c", subcore_axis_name="s")

    nb2 = nb // 2

    @functools.partial(
        pl.kernel,
        out_type=jax.ShapeDtypeStruct((nc, n_pad, hw), _F32),
        mesh=mesh,
        compiler_params=pltpu.CompilerParams(use_tc_tiling_on_sc=False),
        scratch_types=[
            pltpu.VMEM((epw,), jnp.int32),
            pltpu.VMEM((epw,), jnp.int32),
            pltpu.VMEM((2, bsz), jnp.int32),
            pltpu.VMEM((2, bsz, hw), _F32),
            pltpu.VMEM((2, bsz, hw), _F32),
            pltpu.VMEM((zr, hw), _F32),
            pltpu.VMEM((2, d1), _F32),
            pltpu.VMEM_SHARED((n_pad, hw), _F32),
            pltpu.SemaphoreType.DMA,
            pltpu.SemaphoreType.DMA,
            pltpu.SemaphoreType.DMA,
            pltpu.SemaphoreType.DMA,
            pltpu.SemaphoreType.DMA,
            pltpu.SemaphoreType.DMA,
        ],
    )
    def msg(y2_hbm, src_hbm, dst_hbm, h2_hbm, ac_hbm, out_hbm,
            srcall, dstall, didx, rows, y2t, zbuf, acv, aggsh,
            semg0, semg1, semy0, semy1, sems0, sems1):
        cid = lax.axis_index("c")
        sid = lax.axis_index("s")
        semg = (semg0, semg1)
        semy = (semy0, semy1)
        sems = (sems0, sems1)

        def zrow(r, carry):
            for j in range(nlc):
                zbuf[r, pl.ds(j * 16, 16)] = jnp.zeros((16,), _F32)
            return carry

        lax.fori_loop(0, zr, zrow, 0)
        for k in range(rps // zr):
            pltpu.sync_copy(zbuf, aggsh.at[pl.ds(sid * rps + k * zr, zr)])
        pltpu.sync_copy(ac_hbm, acv)

        ebase = sid * epw
        pltpu.sync_copy(src_hbm.at[pl.ds(ebase, epw)], srcall)
        pltpu.sync_copy(dst_hbm.at[pl.ds(ebase, epw)], dstall)
        node_off = cid * n_nodes

        def adj(j, carry):
            sl = pl.ds(j * 16, 16)
            srcall[sl] = srcall[sl] + node_off
            return carry

        lax.fori_loop(0, epw // 16, adj, 0)
        plsc.subcore_barrier()

        a2 = [acv[0, pl.ds(cid * hw + j * 16, 16)] for j in range(nlc)]
        c2 = [acv[1, pl.ds(cid * hw + j * 16, 16)] for j in range(nlc)]
        y2base = cid * e_total + ebase

        def start_slot(slot, i):
            for j in range(nib):
                didx[slot, pl.ds(j * 16, 16)] = \
                    dstall[pl.ds(i * bsz + j * 16, 16)]
            pltpu.async_copy(h2_hbm.at[srcall.at[pl.ds(i * bsz, bsz)]],
                             rows.at[slot], semg[slot])
            pltpu.async_copy(y2_hbm.at[pl.ds(y2base + i * bsz, bsz)],
                             y2t.at[slot], semy[slot])

        def wait_gather(slot, i):
            pltpu.make_async_copy(
                h2_hbm.at[srcall.at[pl.ds(i * bsz, bsz)]],
                rows.at[slot], semg[slot]).wait()
            pltpu.make_async_copy(
                y2_hbm.at[pl.ds(y2base + i * bsz, bsz)],
                y2t.at[slot], semy[slot]).wait()

        def wait_scatter(slot):
            pltpu.make_async_copy(rows.at[slot],
                                  aggsh.at[didx.at[slot]],
                                  sems[slot]).wait()

        def compute(slot):
            def row_body(r, c_):
                for rr in range(2):
                    for j in range(nlc):
                        sl = pl.ds(j * 16, 16)
                        ev = jnp.maximum(
                            y2t[slot, 2 * r + rr, sl] * a2[j] + c2[j], 0.0)
                        rows[slot, 2 * r + rr, sl] = jnp.maximum(
                            rows[slot, 2 * r + rr, sl] + ev, 0.0)
                return c_

            lax.fori_loop(0, bsz // 2, row_body, 0)

        def scatter(slot):
            pltpu.async_copy(rows.at[slot], aggsh.at[didx.at[slot]],
                             sems[slot], add=True)

        start_slot(0, 0)

        def outer(k2, carry):
            i0 = 2 * k2
            wait_gather(0, i0)

            @pl.when(k2 > 0)
            def _():
                wait_scatter(1)

            start_slot(1, i0 + 1)
            compute(0)
            scatter(0)

            wait_gather(1, i0 + 1)

            @pl.when(k2 < nb2 - 1)
            def _():
                wait_scatter(0)
                start_slot(0, i0 + 2)

            compute(1)
            scatter(1)
            return carry

        lax.fori_loop(0, nb2, outer, 0)
        wait_scatter(0)
        wait_scatter(1)
        plsc.subcore_barrier()
        for k in range(rps // zr):
            r0 = sid * rps + k * zr
            pltpu.sync_copy(aggsh.at[pl.ds(r0, zr)],
                            out_hbm.at[cid, pl.ds(r0, zr)])

    return msg


def _node_body(h_ref, agg_ref, pe_ref, w1t_ref, pp_ref, w2t_ref,
               o_ref, os_ref):
    n = h_ref.shape[0]
    pp = pp_ref[...]
    g1 = pp[0:1, :]
    beta1 = pp[1:2, :]
    g2 = pp[2:3, :]
    beta2 = pp[3:4, :]
    agg = jnp.concatenate([agg_ref[0, :n, :], agg_ref[1, :n, :]], axis=1)
    hh = h_ref[...] * pe_ref[...][0:1, :] + agg
    y1 = jnp.dot(hh, w1t_ref[...], preferred_element_type=_F32)
    m1 = jnp.mean(y1, axis=0, keepdims=True)
    v1 = jnp.mean((y1 - m1) ** 2, axis=0, keepdims=True)
    e1 = jnp.maximum(g1 * (y1 - m1) / jnp.sqrt(v1 + 1e-5) + beta1, 0.0)
    y2 = jnp.dot(e1, w2t_ref[...], preferred_element_type=_F32)
    m2 = jnp.mean(y2, axis=0, keepdims=True)
    v2 = jnp.mean((y2 - m2) ** 2, axis=0, keepdims=True)
    o = jnp.maximum(g2 * (y2 - m2) / jnp.sqrt(v2 + 1e-5) + beta2, 0.0)
    o_ref[...] = o
    hw = o.shape[1] // 2
    os_ref[0] = o[:, :hw]
    os_ref[1] = o[:, hw:]


def _node_mlp(h, agg2, c):
    n, d1 = h.shape
    dim = c["mlp_W1"].shape[0]
    pe = jnp.broadcast_to((1.0 + c["eps"][0]) * jnp.ones((1, d1), _F32),
                          (8, d1))
    pp = jnp.concatenate([c["mlp_g1"][None, :], c["mlp_beta1"][None, :],
                          c["mlp_g2"][None, :], c["mlp_beta2"][None, :],
                          jnp.zeros((4, dim), _F32)], axis=0)
    return pl.pallas_call(
        _node_body,
        out_shape=[jax.ShapeDtypeStruct((n, dim), _F32),
                   jax.ShapeDtypeStruct((2, n, dim // 2), _F32)],
    )(h, agg2, pe, c["mlp_W1"].T, pp, c["mlp_W2"].T)


def _s2s_body(h_ref, mask_ref, batch_ref, inter_ref, wiht_ref, whht_ref,
              bias_ref, fc1t_ref, fc4t_ref, o_ref, x_scr, e_scr, ex_scr,
              *, steps, nch, ngraphs, nout):
    _, cs, d = h_ref.shape
    bias = bias_ref[...]
    bgate = bias[0:1, :]
    fc1b = bias[1:2, 0:d]
    fc4b = bias[2:3, 0:1]
    iota_cb = lax.broadcasted_iota(jnp.int32, (cs, ngraphs), 1)

    def fill(k, carry):
        x_scr[k] = h_ref[k] * mask_ref[k]
        return carry

    lax.fori_loop(0, nch, fill, 0)

    q_star = jnp.zeros((ngraphs, 2 * d), _F32)
    h = jnp.zeros((ngraphs, d), _F32)
    c = jnp.zeros((ngraphs, d), _F32)

    for _ in range(steps):
        gates = (jnp.dot(q_star, wiht_ref[...], preferred_element_type=_F32)
                 + jnp.dot(h, whht_ref[...], preferred_element_type=_F32)
                 + bgate)
        ig = gates[:, 0:d]
        fg = gates[:, d:2 * d]
        gg = gates[:, 2 * d:3 * d]
        og = gates[:, 3 * d:4 * d]
        c = jax.nn.sigmoid(fg) * c + jax.nn.sigmoid(ig) * jnp.tanh(gg)
        h = jax.nn.sigmoid(og) * jnp.tanh(c)
        q = h

        def pass1(k, emax):
            oh = (batch_ref[k] == iota_cb).astype(_F32)
            qb = lax.dot_general(oh, q, (((1,), (0,)), ((), ())),
                                 preferred_element_type=_F32)
            ec = jnp.sum(x_scr[k] * qb, axis=1, keepdims=True)
            e_scr[k] = ec
            mk = jnp.max(jnp.where(oh > 0.0, ec, -1e30), axis=0,
                         keepdims=True)
            return jnp.maximum(emax, mk)

        emax = lax.fori_loop(0, nch, pass1,
                             jnp.full((1, ngraphs), -1e30, _F32))
        emax = jnp.where(emax > -1e29, emax, 0.0)

        def pass2(k, denom):
            oh = (batch_ref[k] == iota_cb).astype(_F32)
            epn = jnp.sum(oh * emax, axis=1, keepdims=True)
            ex = jnp.exp(e_scr[k] - epn)
            ex_scr[k] = ex
            return denom + jnp.sum(oh * ex, axis=0, keepdims=True)

        denom = lax.fori_loop(0, nch, pass2, jnp.zeros((1, ngraphs), _F32))

        def pass3(k, r):
            oh = (batch_ref[k] == iota_cb).astype(_F32)
            dpn = jnp.sum(oh * denom, axis=1, keepdims=True)
            a = ex_scr[k] / dpn
            ax = a * x_scr[k]
            return r + lax.dot_general(oh, ax, (((0,), (0,)), ((), ())),
                                       preferred_element_type=_F32)

        r = lax.fori_loop(0, nch, pass3, jnp.zeros((ngraphs, d), _F32))
        q_star = jnp.concatenate([q, r], axis=1)

    h1 = jnp.maximum(
        jnp.dot(q_star, fc1t_ref[...], preferred_element_type=_F32) + fc1b,
        0.0)
    iota_go = lax.broadcasted_iota(jnp.int32, (ngraphs, nout), 1)
    ohi = (inter_ref[...] == iota_go).astype(_F32)
    cnt = lax.dot_general(ohi, jnp.ones((ngraphs, 1), _F32),
                          (((0,), (0,)), ((), ())),
                          preferred_element_type=_F32)
    s = lax.dot_general(ohi, h1, (((0,), (0,)), ((), ())),
                        preferred_element_type=_F32)
    hm = s / jnp.maximum(cnt, 1.0)
    o_ref[...] = (jnp.dot(hm, fc4t_ref[...], preferred_element_type=_F32)
                  + fc4b)


def _set2set_head(h, mask, batch, inter_idx, p, ngraphs, nout):
    n, d = h.shape
    nch = 4
    cs = n // nch
    bias = jnp.zeros((8, 4 * d), _F32)
    bias = bias.at[0, :].set(p["lstm_bih"] + p["lstm_bhh"])
    bias = bias.at[1, 0:d].set(p["fc1_b"])
    bias = bias.at[2, 0].set(p["fc4_b"][0])
    body = functools.partial(_s2s_body, steps=6, nch=nch, ngraphs=ngraphs,
                             nout=nout)
    return pl.pallas_call(
        body,
        out_shape=jax.ShapeDtypeStruct((nout, 1), _F32),
        scratch_shapes=[pltpu.VMEM((nch, cs, d), _F32),
                        pltpu.VMEM((nch, cs, 1), _F32),
                        pltpu.VMEM((nch, cs, 1), _F32)],
    )(h.reshape(nch, cs, d), mask.reshape(nch, cs, 1),
      batch.reshape(nch, cs, 1),
      inter_idx.reshape(ngraphs, 1), p["lstm_Wih"].T, p["lstm_Whh"].T,
      bias, p["fc1_W"].T, p["fc4_W"].T)


def kernel(x, edge_index, edge_attr, extra_feature, selected_node_masks,
           original_node_mask, inter_graph_idx, params):
    n_nodes = x.shape[0]
    e_total = edge_attr.shape[0]
    n_graphs = inter_graph_idx.shape[0]
    n_out = 100
    src = edge_index[0]
    dst = edge_index[1]
    tile = 6400

    h, hs = _encode(x, extra_feature, params)
    mu, gram = _edge_moments(edge_attr, tile)

    for c in params["convs"]:
        w1 = c["be_W1"]
        d1 = w1.shape[0]
        wmu = w1 @ mu
        mean1 = wmu + c["be_b1"]
        ey2 = jnp.einsum("ij,jk,ik->i", w1, gram, w1) \
            + 2.0 * c["be_b1"] * wmu + c["be_b1"] ** 2
        var1 = ey2 - mean1 ** 2
        a1 = c["be_g1"] / jnp.sqrt(var1 + 1e-5)
        c1 = c["be_beta1"] - mean1 * a1

        y2s, s2, q2 = _edge_mlp(edge_attr, w1.T, a1, c1, c["be_b2"],
                                c["be_W2"].T, tile)
        mean2 = s2 / e_total
        var2 = q2 / e_total - mean2 ** 2
        a2 = c["be_g2"] / jnp.sqrt(var2 + 1e-5)
        c2 = c["be_beta2"] - mean2 * a2
        ac = jnp.stack([a2, c2], axis=0)

        msg = _make_msg_kernel(e_total, n_nodes, d1)
        agg2 = msg(y2s.reshape(2 * e_total, d1 // 2), src, dst,
                   hs.reshape(2 * n_nodes, d1 // 2), ac)
        h, hs = _node_mlp(h, agg2, c)

    return _set2set_head(h, selected_node_masks, original_node_mask,
                         inter_graph_idx, params, n_graphs, n_out)
